# Initial kernel scaffold; baseline (speedup 1.0000x reference)
import jax
import jax.numpy as jnp
from jax.experimental import pallas as pl


def kernel(input_batch, input_basis_matrix, center_projection, output_basis, splanifold_anchor_start, splanifold_anchor_end, splanifold_basis_start, splanifold_basis_end, splanifold_pos_tangent_start, splanifold_pos_tangent_end, splanifold_basis_tangent_start, splanifold_basis_tangent_end, splanifold_sigma, splanifold_extrapolation, local_mlp_weight_in, local_mlp_bias_in, local_mlp_weight_out, local_mlp_bias_out, local_mlp_weight_gate, local_mlp_bias_gate, global_mlp_weight_in, global_mlp_bias_in, global_mlp_weight_out, global_mlp_bias_out):
    raise NotImplementedError("write your pallas kernel here")


# speedup vs baseline: 15.3798x; 15.3798x over previous
import jax
import jax.numpy as jnp
from jax.experimental import pallas as pl
from jax.experimental.pallas import tpu as pltpu

N_TOK = 4096
D_IN = 1024
D_OUT = 1024
S = 64
R = 8
P = 3
D_M = 32
K = 6
L_H = 2574
G_H = 256
TEMP = 2.0
SIG_MIN = 0.1
SIG_MAX = 3.0

L_PAD = 2816
H_TOT = L_PAD + G_H
F_DIM = 3328
BK = 256
BN = 1024
BNR = 512
BH = 1024
KH = H_TOT // BK
KF = F_DIM // BK


def _routing_body(x_ref, wib_ref, cen_ref, f_ref, xb_ref):
    x = x_ref[...]
    xs = jnp.where(jnp.isfinite(x), x, 0.0)
    xb_ref[...] = xs.astype(jnp.bfloat16)
    proj = jax.lax.dot_general(
        xs, wib_ref[...], (((1,), (0,)), ((), ())),
        precision=jax.lax.Precision.HIGHEST,
        preferred_element_type=jnp.float32)
    pr = [proj[:, r * S:(r + 1) * S] for r in range(R)]
    cen = cen_ref[...]

    dist = (pr[0] - cen[0:1, :]) ** 2
    for r in range(1, R):
        dist = dist + (pr[r] - cen[r:r + 1, :]) ** 2
    nd = -dist

    bn = nd.shape[0]
    lane = jax.lax.broadcasted_iota(jnp.int32, (bn, S), 1)
    NEG = jnp.float32(-3.0e38)
    work = nd
    chosen = jnp.zeros((bn, S), jnp.bool_)
    for _ in range(K):
        m = jnp.max(work, axis=1, keepdims=True)
        eq = work == m
        idx = jnp.min(jnp.where(eq, lane, S), axis=1, keepdims=True)
        pick = lane == idx
        chosen = jnp.logical_or(chosen, pick)
        work = jnp.where(pick, NEG, work)
    mx = jnp.max(nd, axis=1, keepdims=True)
    z = jnp.where(chosen, jnp.exp((nd - mx) * (1.0 / TEMP)), 0.0)
    gate = z / jnp.sum(z, axis=1, keepdims=True)

    u = [jax.nn.sigmoid(p) for p in pr]
    sum_u = u[0]
    for r in range(1, R):
        sum_u = sum_u + u[r]
    umax = jnp.abs(u[0])
    for r in range(1, R):
        umax = jnp.maximum(umax, jnp.abs(u[r]))
    t = sum_u * (1.0 / R)
    sum_abs = jnp.abs(sum_u)
    sum_eps = jnp.maximum(umax * 0.001, 1e-6)
    fb = sum_abs < sum_eps
    safe = jnp.where(fb, jnp.where(sum_u >= 0, sum_eps, -sum_eps), sum_u)
    w = [jnp.where(fb, 1.0 / R, ur / safe) for ur in u]
    d = [ur - t for ur in u]

    t2 = t * t
    t3 = t2 * t
    h00 = 2.0 * t3 - 3.0 * t2 + 1.0
    h01 = 3.0 * t2 - 2.0 * t3
    h10 = t3 - 2.0 * t2 + t
    h11 = t3 - t2
    c00 = gate * h00
    c01 = gate * h01
    c10 = gate * h10
    c11 = gate * h11

    pieces = [c00, c01, jnp.zeros((bn, 2 * S), jnp.float32)]
    for coef, vec in ((c10, w), (c11, w), (c00, d), (c01, d), (c10, d), (c11, d)):
        for r in range(R):
            pieces.append(coef * vec[r])
    f_ref[...] = jnp.concatenate(pieces, axis=1).astype(jnp.bfloat16)


def _prep_body(sig_ref, a0_ref, a1_ref, b0_ref, b1_ref, pt0_ref, pt1_ref,
               bt0_ref, bt1_ref, ob_ref, p_ref):
    sp = jnp.minimum(jax.nn.softplus(sig_ref[0]) + SIG_MIN, SIG_MAX)
    a0 = a0_ref[0]
    a1 = a1_ref[0]
    b0 = b0_ref[0]
    b1 = b1_ref[0]
    pt0 = pt0_ref[0]
    pt1 = pt1_ref[0]
    bt0 = bt0_ref[0]
    bt1 = bt1_ref[0]
    A0 = jnp.sum(a0, axis=0, keepdims=True)
    A1 = jnp.sum(a1, axis=0, keepdims=True)
    W0 = jnp.zeros((R, D_M), jnp.float32)
    W1 = jnp.zeros((R, D_M), jnp.float32)
    B0 = jnp.zeros((R, D_M), jnp.float32)
    B1 = jnp.zeros((R, D_M), jnp.float32)
    T0 = jnp.zeros((R, D_M), jnp.float32)
    T1 = jnp.zeros((R, D_M), jnp.float32)
    for p in range(P):
        sg = sp[:, p:p + 1]
        sl = slice(p * R, (p + 1) * R)
        W0 = W0 + sg * (pt0[sl] - a0[p:p + 1, :])
        W1 = W1 + sg * (pt1[sl] - a1[p:p + 1, :])
        B0 = B0 + b0[sl]
        B1 = B1 + b1[sl]
        T0 = T0 + sg * (bt0[sl] - b0[sl])
        T1 = T1 + sg * (bt1[sl] - b1[sl])
    M = jnp.concatenate([A0, A1, W0, W1, B0, B1, T0, T1], axis=0)
    p_ref[0] = jnp.dot(M.astype(jnp.bfloat16),
                       ob_ref[0].astype(jnp.bfloat16),
                       preferred_element_type=jnp.float32).astype(jnp.bfloat16)


def _hidden_body(x_ref, wi_ref, wg_ref, bi_ref, bg_ref, h_ref, acc_i, acc_g):
    k = pl.program_id(2)

    @pl.when(k == 0)
    def _():
        acc_i[...] = jnp.zeros_like(acc_i)
        acc_g[...] = jnp.zeros_like(acc_g)

    x = x_ref[...]
    acc_i[...] += jnp.dot(x, wi_ref[...], preferred_element_type=jnp.float32)
    acc_g[...] += jnp.dot(x, wg_ref[...], preferred_element_type=jnp.float32)

    @pl.when(k == pl.num_programs(2) - 1)
    def _():
        hi = jax.nn.gelu(acc_i[...] + bi_ref[...], approximate=True)
        h_ref[...] = (hi * (acc_g[...] + bg_ref[...])).astype(jnp.bfloat16)


def _combine_body(h_ref, f_ref, w_ref, b_ref, o_ref, acc):
    k = pl.program_id(1)

    @pl.when(k == 0)
    def _():
        acc[...] = jnp.zeros_like(acc)

    @pl.when(k < KH)
    def _():
        acc[...] += jnp.dot(h_ref[...], w_ref[...],
                            preferred_element_type=jnp.float32)

    @pl.when(k >= KH)
    def _():
        acc[...] += jnp.dot(f_ref[...], w_ref[...],
                            preferred_element_type=jnp.float32)

    @pl.when(k == pl.num_programs(1) - 1)
    def _():
        o_ref[...] = acc[...] + b_ref[...]


def kernel(input_batch, input_basis_matrix, center_projection, output_basis,
           splanifold_anchor_start, splanifold_anchor_end,
           splanifold_basis_start, splanifold_basis_end,
           splanifold_pos_tangent_start, splanifold_pos_tangent_end,
           splanifold_basis_tangent_start, splanifold_basis_tangent_end,
           splanifold_sigma, splanifold_extrapolation,
           local_mlp_weight_in, local_mlp_bias_in,
           local_mlp_weight_out, local_mlp_bias_out,
           local_mlp_weight_gate, local_mlp_bias_gate,
           global_mlp_weight_in, global_mlp_bias_in,
           global_mlp_weight_out, global_mlp_bias_out):
    f32 = jnp.float32
    bf16 = jnp.bfloat16

    wib = input_basis_matrix.transpose(1, 2, 0).reshape(D_IN, R * S)
    cen = center_projection.T
    f, xb = pl.pallas_call(
        _routing_body,
        grid=(N_TOK // BNR,),
        in_specs=[
            pl.BlockSpec((BNR, D_IN), lambda i: (i, 0)),
            pl.BlockSpec((D_IN, R * S), lambda i: (0, 0)),
            pl.BlockSpec((R, S), lambda i: (0, 0)),
        ],
        out_specs=[
            pl.BlockSpec((BNR, F_DIM), lambda i: (i, 0)),
            pl.BlockSpec((BNR, D_IN), lambda i: (i, 0)),
        ],
        out_shape=[
            jax.ShapeDtypeStruct((N_TOK, F_DIM), bf16),
            jax.ShapeDtypeStruct((N_TOK, D_IN), bf16),
        ],
    )(input_batch, wib, cen)

    sig3 = splanifold_sigma.reshape(S, 1, P)
    b0r = splanifold_basis_start.reshape(S, P * R, D_M)
    b1r = splanifold_basis_end.reshape(S, P * R, D_M)
    pt0r = splanifold_pos_tangent_start.reshape(S, P * R, D_M)
    pt1r = splanifold_pos_tangent_end.reshape(S, P * R, D_M)
    bt0r = splanifold_basis_tangent_start.reshape(S, P * R, D_M)
    bt1r = splanifold_basis_tangent_end.reshape(S, P * R, D_M)
    ptab = pl.pallas_call(
        _prep_body,
        grid=(S,),
        in_specs=[
            pl.BlockSpec((1, 1, P), lambda s: (s, 0, 0)),
            pl.BlockSpec((1, P, D_M), lambda s: (s, 0, 0)),
            pl.BlockSpec((1, P, D_M), lambda s: (s, 0, 0)),
            pl.BlockSpec((1, P * R, D_M), lambda s: (s, 0, 0)),
            pl.BlockSpec((1, P * R, D_M), lambda s: (s, 0, 0)),
            pl.BlockSpec((1, P * R, D_M), lambda s: (s, 0, 0)),
            pl.BlockSpec((1, P * R, D_M), lambda s: (s, 0, 0)),
            pl.BlockSpec((1, P * R, D_M), lambda s: (s, 0, 0)),
            pl.BlockSpec((1, P * R, D_M), lambda s: (s, 0, 0)),
            pl.BlockSpec((1, D_M, D_OUT), lambda s: (s, 0, 0)),
        ],
        out_specs=pl.BlockSpec((1, 50, D_OUT), lambda s: (s, 0, 0)),
        out_shape=jax.ShapeDtypeStruct((S, 50, D_OUT), bf16),
    )(sig3, splanifold_anchor_start, splanifold_anchor_end,
      b0r, b1r, pt0r, pt1r, bt0r, bt1r, output_basis)

    chunks = [ptab[:, 2 + 8 * c:10 + 8 * c, :].transpose(1, 0, 2).reshape(R * S, D_OUT)
              for c in range(6)]
    wc = jnp.concatenate(
        [ptab[:, 0, :], ptab[:, 1, :], jnp.zeros((2 * S, D_OUT), bf16)] + chunks,
        axis=0)

    pad_l = L_PAD - L_H
    wi = jnp.concatenate(
        [jnp.pad(local_mlp_weight_in, ((0, 0), (0, pad_l))),
         global_mlp_weight_in], axis=1).astype(bf16)
    wg = jnp.concatenate(
        [jnp.pad(local_mlp_weight_gate, ((0, 0), (0, pad_l))),
         jnp.zeros((D_IN, G_H), f32)], axis=1).astype(bf16)
    bi = jnp.concatenate(
        [jnp.pad(local_mlp_bias_in, (0, pad_l)), global_mlp_bias_in]
    ).reshape(1, H_TOT)
    bg = jnp.concatenate(
        [jnp.pad(local_mlp_bias_gate, (0, pad_l)), jnp.ones((G_H,), f32)]
    ).reshape(1, H_TOT)
    hid = pl.pallas_call(
        _hidden_body,
        grid=(N_TOK // BN, H_TOT // BH, D_IN // BK),
        in_specs=[
            pl.BlockSpec((BN, BK), lambda n, h, k: (n, k)),
            pl.BlockSpec((BK, BH), lambda n, h, k: (k, h)),
            pl.BlockSpec((BK, BH), lambda n, h, k: (k, h)),
            pl.BlockSpec((1, BH), lambda n, h, k: (0, h)),
            pl.BlockSpec((1, BH), lambda n, h, k: (0, h)),
        ],
        out_specs=pl.BlockSpec((BN, BH), lambda n, h, k: (n, h)),
        out_shape=jax.ShapeDtypeStruct((N_TOK, H_TOT), bf16),
        scratch_shapes=[pltpu.VMEM((BN, BH), f32), pltpu.VMEM((BN, BH), f32)],
    )(xb, wi, wg, bi, bg)

    wo = jnp.concatenate(
        [jnp.pad(local_mlp_weight_out, ((0, pad_l), (0, 0))).astype(bf16),
         global_mlp_weight_out.astype(bf16),
         wc], axis=0)
    btot = (local_mlp_bias_out + global_mlp_bias_out).reshape(1, D_OUT)
    out = pl.pallas_call(
        _combine_body,
        grid=(N_TOK // BN, KH + KF),
        in_specs=[
            pl.BlockSpec((BN, BK), lambda n, k: (n, jnp.minimum(k, KH - 1))),
            pl.BlockSpec((BN, BK), lambda n, k: (n, jnp.maximum(k - KH, 0))),
            pl.BlockSpec((BK, D_OUT), lambda n, k: (k, 0)),
            pl.BlockSpec((1, D_OUT), lambda n, k: (0, 0)),
        ],
        out_specs=pl.BlockSpec((BN, D_OUT), lambda n, k: (n, 0)),
        out_shape=jax.ShapeDtypeStruct((N_TOK, D_OUT), f32),
        scratch_shapes=[pltpu.VMEM((BN, D_OUT), f32)],
    )(hid, f, wo, btot)
    return out


# speedup vs baseline: 20.3764x; 1.3249x over previous
import jax
import jax.numpy as jnp
from jax.experimental import pallas as pl
from jax.experimental.pallas import tpu as pltpu

N_TOK = 4096
D_IN = 1024
D_OUT = 1024
S = 64
R = 8
P = 3
D_M = 32
K = 6
L_H = 2574
G_H = 256
TEMP = 2.0
SIG_MIN = 0.1
SIG_MAX = 3.0

L_PAD = 2816
H_TOT = L_PAD + G_H
F_DIM = 3328
BN = 512
BNR = 512


def _routing_body(x_ref, wib_ref, cen_ref, f_ref, xb_ref):
    x = x_ref[...]
    xs = jnp.where(jnp.isfinite(x), x, 0.0)
    xb_ref[...] = xs.astype(jnp.bfloat16)
    proj = jax.lax.dot_general(
        xs, wib_ref[...], (((1,), (0,)), ((), ())),
        precision=jax.lax.Precision.HIGHEST,
        preferred_element_type=jnp.float32)
    cen = cen_ref[...]

    dist = (proj[:, 0:S] - cen[0:1, :]) ** 2
    for r in range(1, R):
        dist = dist + (proj[:, r * S:(r + 1) * S] - cen[r:r + 1, :]) ** 2
    nd = -dist

    bn = nd.shape[0]
    NEG = jnp.float32(-3.0e38)
    work = nd
    chosen = jnp.zeros((bn, S), jnp.bool_)
    for _ in range(K):
        m = jnp.max(work, axis=1, keepdims=True)
        pick = work == m
        chosen = jnp.logical_or(chosen, pick)
        work = jnp.where(pick, NEG, work)
    mx = jnp.max(nd, axis=1, keepdims=True)
    z = jnp.where(chosen, jnp.exp((nd - mx) * (1.0 / TEMP)), 0.0)
    gate = z / jnp.sum(z, axis=1, keepdims=True)

    u_all = jax.nn.sigmoid(proj)
    ur = [u_all[:, r * S:(r + 1) * S] for r in range(R)]
    sum_u = ur[0]
    for r in range(1, R):
        sum_u = sum_u + ur[r]
    umax = ur[0]
    for r in range(1, R):
        umax = jnp.maximum(umax, ur[r])
    t = sum_u * (1.0 / R)
    sum_eps = jnp.maximum(umax * 0.001, 1e-6)
    fb = jnp.abs(sum_u) < sum_eps
    safe = jnp.where(fb, jnp.where(sum_u >= 0, sum_eps, -sum_eps), sum_u)

    t2 = t * t
    t3 = t2 * t
    h00 = 2.0 * t3 - 3.0 * t2 + 1.0
    h01 = 3.0 * t2 - 2.0 * t3
    h10 = t3 - 2.0 * t2 + t
    h11 = t3 - t2
    c00 = gate * h00
    c01 = gate * h01
    c10 = gate * h10
    c11 = gate * h11

    def tile8(a):
        return jnp.concatenate([a] * R, axis=1)

    fbf = tile8(jnp.where(fb, 1.0, 0.0))
    w_all = fbf * (1.0 / R) + (1.0 - fbf) * (u_all / tile8(safe))
    d_all = u_all - tile8(t)
    c00t = tile8(c00)
    c01t = tile8(c01)
    c10t = tile8(c10)
    c11t = tile8(c11)
    f_ref[...] = jnp.concatenate(
        [c00, c01, jnp.zeros((bn, 2 * S), jnp.float32),
         c10t * w_all, c11t * w_all,
         c00t * d_all, c01t * d_all, c10t * d_all, c11t * d_all],
        axis=1).astype(jnp.bfloat16)


def _prep_body(sig_ref, a0_ref, a1_ref, b0_ref, b1_ref, pt0_ref, pt1_ref,
               bt0_ref, bt1_ref, ob_ref, p_ref):
    sp = jnp.minimum(jax.nn.softplus(sig_ref[0]) + SIG_MIN, SIG_MAX)
    a0 = a0_ref[0]
    a1 = a1_ref[0]
    b0 = b0_ref[0]
    b1 = b1_ref[0]
    pt0 = pt0_ref[0]
    pt1 = pt1_ref[0]
    bt0 = bt0_ref[0]
    bt1 = bt1_ref[0]
    A0 = jnp.sum(a0, axis=0, keepdims=True)
    A1 = jnp.sum(a1, axis=0, keepdims=True)
    W0 = jnp.zeros((R, D_M), jnp.float32)
    W1 = jnp.zeros((R, D_M), jnp.float32)
    B0 = jnp.zeros((R, D_M), jnp.float32)
    B1 = jnp.zeros((R, D_M), jnp.float32)
    T0 = jnp.zeros((R, D_M), jnp.float32)
    T1 = jnp.zeros((R, D_M), jnp.float32)
    for p in range(P):
        sg = sp[:, p:p + 1]
        sl = slice(p * R, (p + 1) * R)
        W0 = W0 + sg * (pt0[sl] - a0[p:p + 1, :])
        W1 = W1 + sg * (pt1[sl] - a1[p:p + 1, :])
        B0 = B0 + b0[sl]
        B1 = B1 + b1[sl]
        T0 = T0 + sg * (bt0[sl] - b0[sl])
        T1 = T1 + sg * (bt1[sl] - b1[sl])
    M = jnp.concatenate([A0, A1, W0, W1, B0, B1, T0, T1], axis=0)
    p_ref[0] = jnp.dot(M.astype(jnp.bfloat16),
                       ob_ref[0].astype(jnp.bfloat16),
                       preferred_element_type=jnp.float32).astype(jnp.bfloat16)


def _hidden_body(x_ref, wi_ref, wg_ref, bi_ref, bg_ref, h_ref):
    x = x_ref[...]
    hi = jnp.dot(x, wi_ref[...], preferred_element_type=jnp.float32)
    hg = jnp.dot(x, wg_ref[...], preferred_element_type=jnp.float32)
    hi = jax.nn.gelu(hi + bi_ref[...], approximate=True)
    h_ref[...] = (hi * (hg + bg_ref[...])).astype(jnp.bfloat16)


def _combine_body(h_ref, f_ref, wh_ref, wf_ref, b_ref, o_ref):
    acc = jnp.dot(h_ref[...], wh_ref[...], preferred_element_type=jnp.float32)
    acc = acc + jnp.dot(f_ref[...], wf_ref[...],
                        preferred_element_type=jnp.float32)
    o_ref[...] = acc + b_ref[...]


def kernel(input_batch, input_basis_matrix, center_projection, output_basis,
           splanifold_anchor_start, splanifold_anchor_end,
           splanifold_basis_start, splanifold_basis_end,
           splanifold_pos_tangent_start, splanifold_pos_tangent_end,
           splanifold_basis_tangent_start, splanifold_basis_tangent_end,
           splanifold_sigma, splanifold_extrapolation,
           local_mlp_weight_in, local_mlp_bias_in,
           local_mlp_weight_out, local_mlp_bias_out,
           local_mlp_weight_gate, local_mlp_bias_gate,
           global_mlp_weight_in, global_mlp_bias_in,
           global_mlp_weight_out, global_mlp_bias_out):
    f32 = jnp.float32
    bf16 = jnp.bfloat16

    wib = input_basis_matrix.transpose(1, 2, 0).reshape(D_IN, R * S)
    cen = center_projection.T
    f, xb = pl.pallas_call(
        _routing_body,
        grid=(N_TOK // BNR,),
        in_specs=[
            pl.BlockSpec((BNR, D_IN), lambda i: (i, 0)),
            pl.BlockSpec((D_IN, R * S), lambda i: (0, 0)),
            pl.BlockSpec((R, S), lambda i: (0, 0)),
        ],
        out_specs=[
            pl.BlockSpec((BNR, F_DIM), lambda i: (i, 0)),
            pl.BlockSpec((BNR, D_IN), lambda i: (i, 0)),
        ],
        out_shape=[
            jax.ShapeDtypeStruct((N_TOK, F_DIM), bf16),
            jax.ShapeDtypeStruct((N_TOK, D_IN), bf16),
        ],
    )(input_batch, wib, cen)

    sig3 = splanifold_sigma.reshape(S, 1, P)
    b0r = splanifold_basis_start.reshape(S, P * R, D_M)
    b1r = splanifold_basis_end.reshape(S, P * R, D_M)
    pt0r = splanifold_pos_tangent_start.reshape(S, P * R, D_M)
    pt1r = splanifold_pos_tangent_end.reshape(S, P * R, D_M)
    bt0r = splanifold_basis_tangent_start.reshape(S, P * R, D_M)
    bt1r = splanifold_basis_tangent_end.reshape(S, P * R, D_M)
    ptab = pl.pallas_call(
        _prep_body,
        grid=(S,),
        in_specs=[
            pl.BlockSpec((1, 1, P), lambda s: (s, 0, 0)),
            pl.BlockSpec((1, P, D_M), lambda s: (s, 0, 0)),
            pl.BlockSpec((1, P, D_M), lambda s: (s, 0, 0)),
            pl.BlockSpec((1, P * R, D_M), lambda s: (s, 0, 0)),
            pl.BlockSpec((1, P * R, D_M), lambda s: (s, 0, 0)),
            pl.BlockSpec((1, P * R, D_M), lambda s: (s, 0, 0)),
            pl.BlockSpec((1, P * R, D_M), lambda s: (s, 0, 0)),
            pl.BlockSpec((1, P * R, D_M), lambda s: (s, 0, 0)),
            pl.BlockSpec((1, P * R, D_M), lambda s: (s, 0, 0)),
            pl.BlockSpec((1, D_M, D_OUT), lambda s: (s, 0, 0)),
        ],
        out_specs=pl.BlockSpec((1, 50, D_OUT), lambda s: (s, 0, 0)),
        out_shape=jax.ShapeDtypeStruct((S, 50, D_OUT), bf16),
    )(sig3, splanifold_anchor_start, splanifold_anchor_end,
      b0r, b1r, pt0r, pt1r, bt0r, bt1r, output_basis)

    chunks = [ptab[:, 2 + 8 * c:10 + 8 * c, :].transpose(1, 0, 2).reshape(R * S, D_OUT)
              for c in range(6)]
    wc = jnp.concatenate(
        [ptab[:, 0, :], ptab[:, 1, :], jnp.zeros((2 * S, D_OUT), bf16)] + chunks,
        axis=0)

    pad_l = L_PAD - L_H
    wi = jnp.concatenate(
        [jnp.pad(local_mlp_weight_in, ((0, 0), (0, pad_l))),
         global_mlp_weight_in], axis=1).astype(bf16)
    wg = jnp.concatenate(
        [jnp.pad(local_mlp_weight_gate, ((0, 0), (0, pad_l))),
         jnp.zeros((D_IN, G_H), f32)], axis=1).astype(bf16)
    bi = jnp.concatenate(
        [jnp.pad(local_mlp_bias_in, (0, pad_l)), global_mlp_bias_in]
    ).reshape(1, H_TOT)
    bg = jnp.concatenate(
        [jnp.pad(local_mlp_bias_gate, (0, pad_l)), jnp.ones((G_H,), f32)]
    ).reshape(1, H_TOT)
    hid = pl.pallas_call(
        _hidden_body,
        grid=(N_TOK // BN,),
        in_specs=[
            pl.BlockSpec((BN, D_IN), lambda n: (n, 0)),
            pl.BlockSpec((D_IN, H_TOT), lambda n: (0, 0)),
            pl.BlockSpec((D_IN, H_TOT), lambda n: (0, 0)),
            pl.BlockSpec((1, H_TOT), lambda n: (0, 0)),
            pl.BlockSpec((1, H_TOT), lambda n: (0, 0)),
        ],
        out_specs=pl.BlockSpec((BN, H_TOT), lambda n: (n, 0)),
        out_shape=jax.ShapeDtypeStruct((N_TOK, H_TOT), bf16),
    )(xb, wi, wg, bi, bg)

    wh = jnp.concatenate(
        [jnp.pad(local_mlp_weight_out, ((0, pad_l), (0, 0))).astype(bf16),
         global_mlp_weight_out.astype(bf16)], axis=0)
    btot = (local_mlp_bias_out + global_mlp_bias_out).reshape(1, D_OUT)
    out = pl.pallas_call(
        _combine_body,
        grid=(N_TOK // BN,),
        in_specs=[
            pl.BlockSpec((BN, H_TOT), lambda n: (n, 0)),
            pl.BlockSpec((BN, F_DIM), lambda n: (n, 0)),
            pl.BlockSpec((H_TOT, D_OUT), lambda n: (0, 0)),
            pl.BlockSpec((F_DIM, D_OUT), lambda n: (0, 0)),
            pl.BlockSpec((1, D_OUT), lambda n: (0, 0)),
        ],
        out_specs=pl.BlockSpec((BN, D_OUT), lambda n: (n, 0)),
        out_shape=jax.ShapeDtypeStruct((N_TOK, D_OUT), f32),
    )(hid, f, wh, wc, btot)
    return out


# speedup vs baseline: 22.0630x; 1.0828x over previous
import jax
import jax.numpy as jnp
from jax.experimental import pallas as pl
from jax.experimental.pallas import tpu as pltpu

N_TOK = 4096
D_IN = 1024
D_OUT = 1024
S = 64
R = 8
P = 3
D_M = 32
K = 6
L_H = 2574
G_H = 256
TEMP = 2.0
SIG_MIN = 0.1
SIG_MAX = 3.0

F_DIM = 3328
BN = 512
BNR = 512


def _prep_body(sig_ref, a0_ref, a1_ref, b0_ref, b1_ref, pt0_ref, pt1_ref,
               bt0_ref, bt1_ref, ob_ref, p_ref):
    sp = jnp.minimum(jax.nn.softplus(sig_ref[0]) + SIG_MIN, SIG_MAX)
    a0 = a0_ref[0]
    a1 = a1_ref[0]
    b0 = b0_ref[0]
    b1 = b1_ref[0]
    pt0 = pt0_ref[0]
    pt1 = pt1_ref[0]
    bt0 = bt0_ref[0]
    bt1 = bt1_ref[0]
    A0 = jnp.sum(a0, axis=0, keepdims=True)
    A1 = jnp.sum(a1, axis=0, keepdims=True)
    W0 = jnp.zeros((R, D_M), jnp.float32)
    W1 = jnp.zeros((R, D_M), jnp.float32)
    B0 = jnp.zeros((R, D_M), jnp.float32)
    B1 = jnp.zeros((R, D_M), jnp.float32)
    T0 = jnp.zeros((R, D_M), jnp.float32)
    T1 = jnp.zeros((R, D_M), jnp.float32)
    for p in range(P):
        sg = sp[:, p:p + 1]
        sl = slice(p * R, (p + 1) * R)
        W0 = W0 + sg * (pt0[sl] - a0[p:p + 1, :])
        W1 = W1 + sg * (pt1[sl] - a1[p:p + 1, :])
        B0 = B0 + b0[sl]
        B1 = B1 + b1[sl]
        T0 = T0 + sg * (bt0[sl] - b0[sl])
        T1 = T1 + sg * (bt1[sl] - b1[sl])
    M = jnp.concatenate([A0, A1, W0, W1, B0, B1, T0, T1], axis=0)
    p_ref[0] = jnp.dot(M.astype(jnp.bfloat16),
                       ob_ref[0].astype(jnp.bfloat16),
                       preferred_element_type=jnp.float32).astype(jnp.bfloat16)


def _routing_body(x_ref, wib_ref, cen_ref, f_ref, xb_ref):
    x = x_ref[...]
    xs = jnp.where(jnp.isfinite(x), x, 0.0)
    xb_ref[...] = xs.astype(jnp.bfloat16)

    proj = jax.lax.dot_general(
        xs, wib_ref[...], (((1,), (0,)), ((), ())),
        precision=jax.lax.Precision.HIGHEST,
        preferred_element_type=jnp.float32)
    cen = cen_ref[...]

    dist = (proj[:, 0:S] - cen[0:1, :]) ** 2
    for r in range(1, R):
        dist = dist + (proj[:, r * S:(r + 1) * S] - cen[r:r + 1, :]) ** 2
    nd = -dist

    bn = nd.shape[0]
    NEG = jnp.float32(-3.0e38)
    work = nd
    chosen = jnp.zeros((bn, S), jnp.bool_)
    for _ in range(K):
        m = jnp.max(work, axis=1, keepdims=True)
        pick = work == m
        chosen = jnp.logical_or(chosen, pick)
        work = jnp.where(pick, NEG, work)
    mx = jnp.max(nd, axis=1, keepdims=True)
    z = jnp.where(chosen, jnp.exp((nd - mx) * (1.0 / TEMP)), 0.0)
    gate = z / jnp.sum(z, axis=1, keepdims=True)

    u_all = jax.nn.sigmoid(proj)
    ur = [u_all[:, r * S:(r + 1) * S] for r in range(R)]
    sum_u = ur[0]
    for r in range(1, R):
        sum_u = sum_u + ur[r]
    umax = ur[0]
    for r in range(1, R):
        umax = jnp.maximum(umax, ur[r])
    t = sum_u * (1.0 / R)
    sum_eps = jnp.maximum(umax * 0.001, 1e-6)
    fb = jnp.abs(sum_u) < sum_eps
    safe = jnp.where(fb, jnp.where(sum_u >= 0, sum_eps, -sum_eps), sum_u)

    t2 = t * t
    t3 = t2 * t
    h00 = 2.0 * t3 - 3.0 * t2 + 1.0
    h01 = 3.0 * t2 - 2.0 * t3
    h10 = t3 - 2.0 * t2 + t
    h11 = t3 - t2
    c00 = gate * h00
    c01 = gate * h01
    c10 = gate * h10
    c11 = gate * h11

    def tile8(a):
        return jnp.concatenate([a] * R, axis=1)

    fbf = tile8(jnp.where(fb, 1.0, 0.0))
    w_all = fbf * (1.0 / R) + (1.0 - fbf) * (u_all / tile8(safe))
    d_all = u_all - tile8(t)
    c00t = tile8(c00)
    c01t = tile8(c01)
    c10t = tile8(c10)
    c11t = tile8(c11)
    f_ref[...] = jnp.concatenate(
        [c00, c01, jnp.zeros((bn, 2 * S), jnp.float32),
         c10t * w_all, c11t * w_all,
         c00t * d_all, c01t * d_all, c10t * d_all, c11t * d_all],
        axis=1).astype(jnp.bfloat16)


def _hidden_body(x_ref, wil_ref, wgl_ref, bil_ref, bgl_ref, wgi_ref, bgi_ref,
                 hl_ref, gg_ref):
    xb = x_ref[...]
    hi = jnp.dot(xb, wil_ref[...], preferred_element_type=jnp.float32)
    hg = jnp.dot(xb, wgl_ref[...], preferred_element_type=jnp.float32)
    hl_ref[...] = (jax.nn.gelu(hi + bil_ref[...], approximate=True)
                   * (hg + bgl_ref[...])).astype(jnp.bfloat16)
    gg_ref[...] = jax.nn.gelu(
        jnp.dot(xb, wgi_ref[...], preferred_element_type=jnp.float32)
        + bgi_ref[...], approximate=True).astype(jnp.bfloat16)


def _combine_body(hl_ref, gg_ref, f_ref, wol_ref, wog_ref, wc_ref, b_ref,
                  o_ref):
    acc = jnp.dot(hl_ref[...], wol_ref[...],
                  preferred_element_type=jnp.float32)
    acc = acc + jnp.dot(gg_ref[...], wog_ref[...],
                        preferred_element_type=jnp.float32)
    acc = acc + jnp.dot(f_ref[...], wc_ref[...],
                        preferred_element_type=jnp.float32)
    o_ref[...] = acc + b_ref[...]


def kernel(input_batch, input_basis_matrix, center_projection, output_basis,
           splanifold_anchor_start, splanifold_anchor_end,
           splanifold_basis_start, splanifold_basis_end,
           splanifold_pos_tangent_start, splanifold_pos_tangent_end,
           splanifold_basis_tangent_start, splanifold_basis_tangent_end,
           splanifold_sigma, splanifold_extrapolation,
           local_mlp_weight_in, local_mlp_bias_in,
           local_mlp_weight_out, local_mlp_bias_out,
           local_mlp_weight_gate, local_mlp_bias_gate,
           global_mlp_weight_in, global_mlp_bias_in,
           global_mlp_weight_out, global_mlp_bias_out):
    f32 = jnp.float32
    bf16 = jnp.bfloat16

    sig3 = splanifold_sigma.reshape(S, 1, P)
    b0r = splanifold_basis_start.reshape(S, P * R, D_M)
    b1r = splanifold_basis_end.reshape(S, P * R, D_M)
    pt0r = splanifold_pos_tangent_start.reshape(S, P * R, D_M)
    pt1r = splanifold_pos_tangent_end.reshape(S, P * R, D_M)
    bt0r = splanifold_basis_tangent_start.reshape(S, P * R, D_M)
    bt1r = splanifold_basis_tangent_end.reshape(S, P * R, D_M)
    ptab = pl.pallas_call(
        _prep_body,
        grid=(S,),
        in_specs=[
            pl.BlockSpec((1, 1, P), lambda s: (s, 0, 0)),
            pl.BlockSpec((1, P, D_M), lambda s: (s, 0, 0)),
            pl.BlockSpec((1, P, D_M), lambda s: (s, 0, 0)),
            pl.BlockSpec((1, P * R, D_M), lambda s: (s, 0, 0)),
            pl.BlockSpec((1, P * R, D_M), lambda s: (s, 0, 0)),
            pl.BlockSpec((1, P * R, D_M), lambda s: (s, 0, 0)),
            pl.BlockSpec((1, P * R, D_M), lambda s: (s, 0, 0)),
            pl.BlockSpec((1, P * R, D_M), lambda s: (s, 0, 0)),
            pl.BlockSpec((1, P * R, D_M), lambda s: (s, 0, 0)),
            pl.BlockSpec((1, D_M, D_OUT), lambda s: (s, 0, 0)),
        ],
        out_specs=pl.BlockSpec((1, 50, D_OUT), lambda s: (s, 0, 0)),
        out_shape=jax.ShapeDtypeStruct((S, 50, D_OUT), bf16),
    )(sig3, splanifold_anchor_start, splanifold_anchor_end,
      b0r, b1r, pt0r, pt1r, bt0r, bt1r, output_basis)

    chunks = [ptab[:, 2 + 8 * c:10 + 8 * c, :].transpose(1, 0, 2).reshape(R * S, D_OUT)
              for c in range(6)]
    wc = jnp.concatenate(
        [ptab[:, 0, :], ptab[:, 1, :], jnp.zeros((2 * S, D_OUT), bf16)] + chunks,
        axis=0)

    wib = input_basis_matrix.transpose(1, 2, 0).reshape(D_IN, R * S)
    cen = center_projection.T
    full = lambda shape: pl.BlockSpec(shape, lambda i: tuple(0 for _ in shape))
    f, xb = pl.pallas_call(
        _routing_body,
        grid=(N_TOK // BNR,),
        in_specs=[
            pl.BlockSpec((BNR, D_IN), lambda i: (i, 0)),
            full((D_IN, R * S)),
            full((R, S)),
        ],
        out_specs=[
            pl.BlockSpec((BNR, F_DIM), lambda i: (i, 0)),
            pl.BlockSpec((BNR, D_IN), lambda i: (i, 0)),
        ],
        out_shape=[
            jax.ShapeDtypeStruct((N_TOK, F_DIM), bf16),
            jax.ShapeDtypeStruct((N_TOK, D_IN), bf16),
        ],
    )(input_batch, wib, cen)

    hl, gg = pl.pallas_call(
        _hidden_body,
        grid=(N_TOK // BN,),
        in_specs=[
            pl.BlockSpec((BN, D_IN), lambda n: (n, 0)),
            full((D_IN, L_H)),
            full((D_IN, L_H)),
            full((1, L_H)),
            full((1, L_H)),
            full((D_IN, G_H)),
            full((1, G_H)),
        ],
        out_specs=[
            pl.BlockSpec((BN, L_H), lambda n: (n, 0)),
            pl.BlockSpec((BN, G_H), lambda n: (n, 0)),
        ],
        out_shape=[
            jax.ShapeDtypeStruct((N_TOK, L_H), bf16),
            jax.ShapeDtypeStruct((N_TOK, G_H), bf16),
        ],
    )(xb, local_mlp_weight_in.astype(bf16), local_mlp_weight_gate.astype(bf16),
      local_mlp_bias_in.reshape(1, L_H), local_mlp_bias_gate.reshape(1, L_H),
      global_mlp_weight_in.astype(bf16), global_mlp_bias_in.reshape(1, G_H))

    out = pl.pallas_call(
        _combine_body,
        grid=(N_TOK // BN,),
        in_specs=[
            pl.BlockSpec((BN, L_H), lambda n: (n, 0)),
            pl.BlockSpec((BN, G_H), lambda n: (n, 0)),
            pl.BlockSpec((BN, F_DIM), lambda n: (n, 0)),
            full((L_H, D_OUT)),
            full((G_H, D_OUT)),
            full((F_DIM, D_OUT)),
            full((1, D_OUT)),
        ],
        out_specs=pl.BlockSpec((BN, D_OUT), lambda n: (n, 0)),
        out_shape=jax.ShapeDtypeStruct((N_TOK, D_OUT), f32),
    )(hl, gg, f, local_mlp_weight_out.astype(bf16),
      global_mlp_weight_out.astype(bf16), wc,
      (local_mlp_bias_out + global_mlp_bias_out).reshape(1, D_OUT))
    return out


# speedup vs baseline: 26.4449x; 1.1986x over previous
import jax
import jax.numpy as jnp
from jax.experimental import pallas as pl
from jax.experimental.pallas import tpu as pltpu

N_TOK = 4096
D_IN = 1024
D_OUT = 1024
S = 64
R = 8
P = 3
D_M = 32
K = 6
L_H = 2574
G_H = 256
TEMP = 2.0
SIG_MIN = 0.1
SIG_MAX = 3.0

F_DIM = 3328
BN = 512
BNR = 512


SB = 8


def _prep_body(sig_ref, a0_ref, a1_ref, b0_ref, b1_ref, pt0_ref, pt1_ref,
               bt0_ref, bt1_ref, ob_ref, p_ref):
    for i in range(SB):
        sp = jnp.minimum(jax.nn.softplus(sig_ref[i]) + SIG_MIN, SIG_MAX)
        a0 = a0_ref[i]
        a1 = a1_ref[i]
        b0 = b0_ref[i]
        b1 = b1_ref[i]
        pt0 = pt0_ref[i]
        pt1 = pt1_ref[i]
        bt0 = bt0_ref[i]
        bt1 = bt1_ref[i]
        A0 = jnp.sum(a0, axis=0, keepdims=True)
        A1 = jnp.sum(a1, axis=0, keepdims=True)
        W0 = jnp.zeros((R, D_M), jnp.float32)
        W1 = jnp.zeros((R, D_M), jnp.float32)
        B0 = jnp.zeros((R, D_M), jnp.float32)
        B1 = jnp.zeros((R, D_M), jnp.float32)
        T0 = jnp.zeros((R, D_M), jnp.float32)
        T1 = jnp.zeros((R, D_M), jnp.float32)
        for p in range(P):
            sg = sp[:, p:p + 1]
            sl = slice(p * R, (p + 1) * R)
            W0 = W0 + sg * (pt0[sl] - a0[p:p + 1, :])
            W1 = W1 + sg * (pt1[sl] - a1[p:p + 1, :])
            B0 = B0 + b0[sl]
            B1 = B1 + b1[sl]
            T0 = T0 + sg * (bt0[sl] - b0[sl])
            T1 = T1 + sg * (bt1[sl] - b1[sl])
        M = jnp.concatenate([A0, A1, W0, W1, B0, B1, T0, T1], axis=0)
        p_ref[i] = jnp.dot(M.astype(jnp.bfloat16),
                           ob_ref[i].astype(jnp.bfloat16),
                           preferred_element_type=jnp.float32).astype(jnp.bfloat16)


def _routing_body(x_ref, wib_ref, cen_ref, f_ref, xb_ref):
    x = x_ref[...]
    xs = jnp.where(jnp.isfinite(x), x, 0.0)
    xb_ref[...] = xs.astype(jnp.bfloat16)

    proj = jax.lax.dot_general(
        xs, wib_ref[...], (((1,), (0,)), ((), ())),
        preferred_element_type=jnp.float32)
    cen = cen_ref[...]

    dist = (proj[:, 0:S] - cen[0:1, :]) ** 2
    for r in range(1, R):
        dist = dist + (proj[:, r * S:(r + 1) * S] - cen[r:r + 1, :]) ** 2
    nd = -dist

    bn = nd.shape[0]
    NEG = jnp.float32(-3.0e38)
    work = nd
    chosen = jnp.zeros((bn, S), jnp.bool_)
    for _ in range(K):
        m = jnp.max(work, axis=1, keepdims=True)
        pick = work == m
        chosen = jnp.logical_or(chosen, pick)
        work = jnp.where(pick, NEG, work)
    mx = jnp.max(nd, axis=1, keepdims=True)
    z = jnp.where(chosen, jnp.exp((nd - mx) * (1.0 / TEMP)), 0.0)
    gate = z / jnp.sum(z, axis=1, keepdims=True)

    u_all = jax.nn.sigmoid(proj)
    ur = [u_all[:, r * S:(r + 1) * S] for r in range(R)]
    sum_u = ur[0]
    for r in range(1, R):
        sum_u = sum_u + ur[r]
    umax = ur[0]
    for r in range(1, R):
        umax = jnp.maximum(umax, ur[r])
    t = sum_u * (1.0 / R)
    sum_eps = jnp.maximum(umax * 0.001, 1e-6)
    fb = jnp.abs(sum_u) < sum_eps
    safe = jnp.where(fb, jnp.where(sum_u >= 0, sum_eps, -sum_eps), sum_u)

    t2 = t * t
    t3 = t2 * t
    h00 = 2.0 * t3 - 3.0 * t2 + 1.0
    h01 = 3.0 * t2 - 2.0 * t3
    h10 = t3 - 2.0 * t2 + t
    h11 = t3 - t2
    c00 = gate * h00
    c01 = gate * h01
    c10 = gate * h10
    c11 = gate * h11

    def tile8(a):
        return jnp.concatenate([a] * R, axis=1)

    fbf = tile8(jnp.where(fb, 1.0, 0.0))
    w_all = fbf * (1.0 / R) + (1.0 - fbf) * (u_all / tile8(safe))
    d_all = u_all - tile8(t)
    c00t = tile8(c00)
    c01t = tile8(c01)
    c10t = tile8(c10)
    c11t = tile8(c11)
    f_ref[...] = jnp.concatenate(
        [c00, c01, jnp.zeros((bn, 2 * S), jnp.float32),
         c10t * w_all, c11t * w_all,
         c00t * d_all, c01t * d_all, c10t * d_all, c11t * d_all],
        axis=1).astype(jnp.bfloat16)


def _hidden_body(x_ref, wil_ref, wgl_ref, bil_ref, bgl_ref, wgi_ref, bgi_ref,
                 hl_ref, gg_ref):
    xb = x_ref[...]
    hi = jnp.dot(xb, wil_ref[...], preferred_element_type=jnp.float32)
    hg = jnp.dot(xb, wgl_ref[...], preferred_element_type=jnp.float32)
    hl_ref[...] = (jax.nn.gelu(hi + bil_ref[...], approximate=True)
                   * (hg + bgl_ref[...])).astype(jnp.bfloat16)
    gg_ref[...] = jax.nn.gelu(
        jnp.dot(xb, wgi_ref[...], preferred_element_type=jnp.float32)
        + bgi_ref[...], approximate=True).astype(jnp.bfloat16)


def _combine_body(hl_ref, gg_ref, f_ref, wol_ref, wog_ref, wc_ref, b_ref,
                  o_ref):
    acc = jnp.dot(hl_ref[...], wol_ref[...],
                  preferred_element_type=jnp.float32)
    acc = acc + jnp.dot(gg_ref[...], wog_ref[...],
                        preferred_element_type=jnp.float32)
    acc = acc + jnp.dot(f_ref[...], wc_ref[...],
                        preferred_element_type=jnp.float32)
    o_ref[...] = acc + b_ref[...]


def kernel(input_batch, input_basis_matrix, center_projection, output_basis,
           splanifold_anchor_start, splanifold_anchor_end,
           splanifold_basis_start, splanifold_basis_end,
           splanifold_pos_tangent_start, splanifold_pos_tangent_end,
           splanifold_basis_tangent_start, splanifold_basis_tangent_end,
           splanifold_sigma, splanifold_extrapolation,
           local_mlp_weight_in, local_mlp_bias_in,
           local_mlp_weight_out, local_mlp_bias_out,
           local_mlp_weight_gate, local_mlp_bias_gate,
           global_mlp_weight_in, global_mlp_bias_in,
           global_mlp_weight_out, global_mlp_bias_out):
    f32 = jnp.float32
    bf16 = jnp.bfloat16

    sig3 = splanifold_sigma.reshape(S, 1, P)
    b0r = splanifold_basis_start.reshape(S, P * R, D_M)
    b1r = splanifold_basis_end.reshape(S, P * R, D_M)
    pt0r = splanifold_pos_tangent_start.reshape(S, P * R, D_M)
    pt1r = splanifold_pos_tangent_end.reshape(S, P * R, D_M)
    bt0r = splanifold_basis_tangent_start.reshape(S, P * R, D_M)
    bt1r = splanifold_basis_tangent_end.reshape(S, P * R, D_M)
    ptab = pl.pallas_call(
        _prep_body,
        grid=(S // SB,),
        in_specs=[
            pl.BlockSpec((SB, 1, P), lambda s: (s, 0, 0)),
            pl.BlockSpec((SB, P, D_M), lambda s: (s, 0, 0)),
            pl.BlockSpec((SB, P, D_M), lambda s: (s, 0, 0)),
            pl.BlockSpec((SB, P * R, D_M), lambda s: (s, 0, 0)),
            pl.BlockSpec((SB, P * R, D_M), lambda s: (s, 0, 0)),
            pl.BlockSpec((SB, P * R, D_M), lambda s: (s, 0, 0)),
            pl.BlockSpec((SB, P * R, D_M), lambda s: (s, 0, 0)),
            pl.BlockSpec((SB, P * R, D_M), lambda s: (s, 0, 0)),
            pl.BlockSpec((SB, P * R, D_M), lambda s: (s, 0, 0)),
            pl.BlockSpec((SB, D_M, D_OUT), lambda s: (s, 0, 0)),
        ],
        out_specs=pl.BlockSpec((SB, 50, D_OUT), lambda s: (s, 0, 0)),
        out_shape=jax.ShapeDtypeStruct((S, 50, D_OUT), bf16),
    )(sig3, splanifold_anchor_start, splanifold_anchor_end,
      b0r, b1r, pt0r, pt1r, bt0r, bt1r, output_basis)

    chunks = [ptab[:, 2 + 8 * c:10 + 8 * c, :].transpose(1, 0, 2).reshape(R * S, D_OUT)
              for c in range(6)]
    wc = jnp.concatenate(
        [ptab[:, 0, :], ptab[:, 1, :], jnp.zeros((2 * S, D_OUT), bf16)] + chunks,
        axis=0)

    wib = input_basis_matrix.transpose(1, 2, 0).reshape(D_IN, R * S)
    cen = center_projection.T
    full = lambda shape: pl.BlockSpec(shape, lambda i: tuple(0 for _ in shape))
    f, xb = pl.pallas_call(
        _routing_body,
        grid=(N_TOK // BNR,),
        in_specs=[
            pl.BlockSpec((BNR, D_IN), lambda i: (i, 0)),
            full((D_IN, R * S)),
            full((R, S)),
        ],
        out_specs=[
            pl.BlockSpec((BNR, F_DIM), lambda i: (i, 0)),
            pl.BlockSpec((BNR, D_IN), lambda i: (i, 0)),
        ],
        out_shape=[
            jax.ShapeDtypeStruct((N_TOK, F_DIM), bf16),
            jax.ShapeDtypeStruct((N_TOK, D_IN), bf16),
        ],
    )(input_batch, wib, cen)

    hl, gg = pl.pallas_call(
        _hidden_body,
        grid=(N_TOK // BN,),
        in_specs=[
            pl.BlockSpec((BN, D_IN), lambda n: (n, 0)),
            full((D_IN, L_H)),
            full((D_IN, L_H)),
            full((1, L_H)),
            full((1, L_H)),
            full((D_IN, G_H)),
            full((1, G_H)),
        ],
        out_specs=[
            pl.BlockSpec((BN, L_H), lambda n: (n, 0)),
            pl.BlockSpec((BN, G_H), lambda n: (n, 0)),
        ],
        out_shape=[
            jax.ShapeDtypeStruct((N_TOK, L_H), bf16),
            jax.ShapeDtypeStruct((N_TOK, G_H), bf16),
        ],
    )(xb, local_mlp_weight_in.astype(bf16), local_mlp_weight_gate.astype(bf16),
      local_mlp_bias_in.reshape(1, L_H), local_mlp_bias_gate.reshape(1, L_H),
      global_mlp_weight_in.astype(bf16), global_mlp_bias_in.reshape(1, G_H))

    out = pl.pallas_call(
        _combine_body,
        grid=(N_TOK // BN,),
        in_specs=[
            pl.BlockSpec((BN, L_H), lambda n: (n, 0)),
            pl.BlockSpec((BN, G_H), lambda n: (n, 0)),
            pl.BlockSpec((BN, F_DIM), lambda n: (n, 0)),
            full((L_H, D_OUT)),
            full((G_H, D_OUT)),
            full((F_DIM, D_OUT)),
            full((1, D_OUT)),
        ],
        out_specs=pl.BlockSpec((BN, D_OUT), lambda n: (n, 0)),
        out_shape=jax.ShapeDtypeStruct((N_TOK, D_OUT), f32),
    )(hl, gg, f, local_mlp_weight_out.astype(bf16),
      global_mlp_weight_out.astype(bf16), wc,
      (local_mlp_bias_out + global_mlp_bias_out).reshape(1, D_OUT))
    return out


# speedup vs baseline: 27.0493x; 1.0229x over previous
import jax
import jax.numpy as jnp
from jax.experimental import pallas as pl
from jax.experimental.pallas import tpu as pltpu

N_TOK = 4096
D_IN = 1024
D_OUT = 1024
S = 64
R = 8
P = 3
D_M = 32
K = 6
L_H = 2574
G_H = 256
TEMP = 2.0
SIG_MIN = 0.1
SIG_MAX = 3.0

F_DIM = 3328
BN = 512
BNR = 512


SB = 8


def _prep_body(sig_ref, a0_ref, a1_ref, b0_ref, b1_ref, pt0_ref, pt1_ref,
               bt0_ref, bt1_ref, ob_ref, p_ref):
    for i in range(SB):
        sp = jnp.minimum(jax.nn.softplus(sig_ref[i]) + SIG_MIN, SIG_MAX)
        a0 = a0_ref[i]
        a1 = a1_ref[i]
        b0 = b0_ref[i]
        b1 = b1_ref[i]
        pt0 = pt0_ref[i]
        pt1 = pt1_ref[i]
        bt0 = bt0_ref[i]
        bt1 = bt1_ref[i]
        A0 = jnp.sum(a0, axis=0, keepdims=True)
        A1 = jnp.sum(a1, axis=0, keepdims=True)
        W0 = jnp.zeros((R, D_M), jnp.float32)
        W1 = jnp.zeros((R, D_M), jnp.float32)
        B0 = jnp.zeros((R, D_M), jnp.float32)
        B1 = jnp.zeros((R, D_M), jnp.float32)
        T0 = jnp.zeros((R, D_M), jnp.float32)
        T1 = jnp.zeros((R, D_M), jnp.float32)
        for p in range(P):
            sg = sp[:, p:p + 1]
            sl = slice(p * R, (p + 1) * R)
            W0 = W0 + sg * (pt0[sl] - a0[p:p + 1, :])
            W1 = W1 + sg * (pt1[sl] - a1[p:p + 1, :])
            B0 = B0 + b0[sl]
            B1 = B1 + b1[sl]
            T0 = T0 + sg * (bt0[sl] - b0[sl])
            T1 = T1 + sg * (bt1[sl] - b1[sl])
        M = jnp.concatenate([A0, A1, W0, W1, B0, B1, T0, T1], axis=0)
        p_ref[i] = jnp.dot(M.astype(jnp.bfloat16),
                           ob_ref[i].astype(jnp.bfloat16),
                           preferred_element_type=jnp.float32).astype(jnp.bfloat16)


def _routing_body(x_ref, wib_ref, cen_ref, f_ref, xb_ref):
    x = x_ref[...]
    xs = jnp.where(jnp.isfinite(x), x, 0.0)
    xb_ref[...] = xs.astype(jnp.bfloat16)

    proj = jax.lax.dot_general(
        xs, wib_ref[...], (((1,), (0,)), ((), ())),
        preferred_element_type=jnp.float32)
    cen = cen_ref[...]

    dist = (proj[:, 0:S] - cen[0:1, :]) ** 2
    for r in range(1, R):
        dist = dist + (proj[:, r * S:(r + 1) * S] - cen[r:r + 1, :]) ** 2
    nd = -dist

    bn = nd.shape[0]
    NEG = jnp.float32(-3.0e38)
    work = nd
    chosen = jnp.zeros((bn, S), jnp.bool_)
    for _ in range(K):
        m = jnp.max(work, axis=1, keepdims=True)
        pick = work == m
        chosen = jnp.logical_or(chosen, pick)
        work = jnp.where(pick, NEG, work)
    mx = jnp.max(nd, axis=1, keepdims=True)
    z = jnp.where(chosen, jnp.exp((nd - mx) * (1.0 / TEMP)), 0.0)
    gate = z / jnp.sum(z, axis=1, keepdims=True)

    u_all = jax.nn.sigmoid(proj)
    ur = [u_all[:, r * S:(r + 1) * S] for r in range(R)]
    sum_u = ur[0]
    for r in range(1, R):
        sum_u = sum_u + ur[r]
    umax = ur[0]
    for r in range(1, R):
        umax = jnp.maximum(umax, ur[r])
    t = sum_u * (1.0 / R)
    sum_eps = jnp.maximum(umax * 0.001, 1e-6)
    fb = jnp.abs(sum_u) < sum_eps
    safe = jnp.where(fb, jnp.where(sum_u >= 0, sum_eps, -sum_eps), sum_u)

    t2 = t * t
    t3 = t2 * t
    h00 = 2.0 * t3 - 3.0 * t2 + 1.0
    h01 = 3.0 * t2 - 2.0 * t3
    h10 = t3 - 2.0 * t2 + t
    h11 = t3 - t2
    bf = jnp.bfloat16
    c00 = (gate * h00).astype(bf)
    c01 = (gate * h01).astype(bf)
    c10 = (gate * h10).astype(bf)
    c11 = (gate * h11).astype(bf)

    def tile8(a):
        return jnp.concatenate([a] * R, axis=1)

    u_b = u_all.astype(bf)
    fbf = tile8(jnp.where(fb, 1.0, 0.0).astype(bf))
    w_all = fbf * bf(1.0 / R) + (bf(1.0) - fbf) * (u_b / tile8(safe.astype(bf)))
    d_all = u_b - tile8(t.astype(bf))
    c00t = tile8(c00)
    c01t = tile8(c01)
    c10t = tile8(c10)
    c11t = tile8(c11)
    f_ref[...] = jnp.concatenate(
        [c00, c01, jnp.zeros((bn, 2 * S), bf),
         c10t * w_all, c11t * w_all,
         c00t * d_all, c01t * d_all, c10t * d_all, c11t * d_all],
        axis=1)


def _hidden_body(x_ref, wil_ref, wgl_ref, bil_ref, bgl_ref, wgi_ref, bgi_ref,
                 hl_ref, gg_ref):
    xb = x_ref[...]
    hi = jnp.dot(xb, wil_ref[...],
                 preferred_element_type=jnp.float32).astype(jnp.bfloat16)
    hg = jnp.dot(xb, wgl_ref[...],
                 preferred_element_type=jnp.float32).astype(jnp.bfloat16)
    hl_ref[...] = (jax.nn.gelu(hi + bil_ref[...], approximate=True)
                   * (hg + bgl_ref[...]))
    gg_ref[...] = jax.nn.gelu(
        jnp.dot(xb, wgi_ref[...],
                preferred_element_type=jnp.float32).astype(jnp.bfloat16)
        + bgi_ref[...], approximate=True)


def _combine_body(hl_ref, gg_ref, f_ref, wol_ref, wog_ref, wc_ref, b_ref,
                  o_ref):
    acc = jnp.dot(hl_ref[...], wol_ref[...],
                  preferred_element_type=jnp.float32)
    acc = acc + jnp.dot(gg_ref[...], wog_ref[...],
                        preferred_element_type=jnp.float32)
    acc = acc + jnp.dot(f_ref[...], wc_ref[...],
                        preferred_element_type=jnp.float32)
    o_ref[...] = acc + b_ref[...]


def kernel(input_batch, input_basis_matrix, center_projection, output_basis,
           splanifold_anchor_start, splanifold_anchor_end,
           splanifold_basis_start, splanifold_basis_end,
           splanifold_pos_tangent_start, splanifold_pos_tangent_end,
           splanifold_basis_tangent_start, splanifold_basis_tangent_end,
           splanifold_sigma, splanifold_extrapolation,
           local_mlp_weight_in, local_mlp_bias_in,
           local_mlp_weight_out, local_mlp_bias_out,
           local_mlp_weight_gate, local_mlp_bias_gate,
           global_mlp_weight_in, global_mlp_bias_in,
           global_mlp_weight_out, global_mlp_bias_out):
    f32 = jnp.float32
    bf16 = jnp.bfloat16

    sig3 = splanifold_sigma.reshape(S, 1, P)
    b0r = splanifold_basis_start.reshape(S, P * R, D_M)
    b1r = splanifold_basis_end.reshape(S, P * R, D_M)
    pt0r = splanifold_pos_tangent_start.reshape(S, P * R, D_M)
    pt1r = splanifold_pos_tangent_end.reshape(S, P * R, D_M)
    bt0r = splanifold_basis_tangent_start.reshape(S, P * R, D_M)
    bt1r = splanifold_basis_tangent_end.reshape(S, P * R, D_M)
    ptab = pl.pallas_call(
        _prep_body,
        grid=(S // SB,),
        in_specs=[
            pl.BlockSpec((SB, 1, P), lambda s: (s, 0, 0)),
            pl.BlockSpec((SB, P, D_M), lambda s: (s, 0, 0)),
            pl.BlockSpec((SB, P, D_M), lambda s: (s, 0, 0)),
            pl.BlockSpec((SB, P * R, D_M), lambda s: (s, 0, 0)),
            pl.BlockSpec((SB, P * R, D_M), lambda s: (s, 0, 0)),
            pl.BlockSpec((SB, P * R, D_M), lambda s: (s, 0, 0)),
            pl.BlockSpec((SB, P * R, D_M), lambda s: (s, 0, 0)),
            pl.BlockSpec((SB, P * R, D_M), lambda s: (s, 0, 0)),
            pl.BlockSpec((SB, P * R, D_M), lambda s: (s, 0, 0)),
            pl.BlockSpec((SB, D_M, D_OUT), lambda s: (s, 0, 0)),
        ],
        out_specs=pl.BlockSpec((SB, 50, D_OUT), lambda s: (s, 0, 0)),
        out_shape=jax.ShapeDtypeStruct((S, 50, D_OUT), bf16),
    )(sig3, splanifold_anchor_start, splanifold_anchor_end,
      b0r, b1r, pt0r, pt1r, bt0r, bt1r, output_basis)

    chunks = [ptab[:, 2 + 8 * c:10 + 8 * c, :].transpose(1, 0, 2).reshape(R * S, D_OUT)
              for c in range(6)]
    wc = jnp.concatenate(
        [ptab[:, 0, :], ptab[:, 1, :], jnp.zeros((2 * S, D_OUT), bf16)] + chunks,
        axis=0)

    wib = input_basis_matrix.transpose(1, 2, 0).reshape(D_IN, R * S)
    cen = center_projection.T
    full = lambda shape: pl.BlockSpec(shape, lambda i: tuple(0 for _ in shape))
    f, xb = pl.pallas_call(
        _routing_body,
        grid=(N_TOK // BNR,),
        in_specs=[
            pl.BlockSpec((BNR, D_IN), lambda i: (i, 0)),
            full((D_IN, R * S)),
            full((R, S)),
        ],
        out_specs=[
            pl.BlockSpec((BNR, F_DIM), lambda i: (i, 0)),
            pl.BlockSpec((BNR, D_IN), lambda i: (i, 0)),
        ],
        out_shape=[
            jax.ShapeDtypeStruct((N_TOK, F_DIM), bf16),
            jax.ShapeDtypeStruct((N_TOK, D_IN), bf16),
        ],
    )(input_batch, wib, cen)

    hl, gg = pl.pallas_call(
        _hidden_body,
        grid=(N_TOK // BN,),
        in_specs=[
            pl.BlockSpec((BN, D_IN), lambda n: (n, 0)),
            full((D_IN, L_H)),
            full((D_IN, L_H)),
            full((1, L_H)),
            full((1, L_H)),
            full((D_IN, G_H)),
            full((1, G_H)),
        ],
        out_specs=[
            pl.BlockSpec((BN, L_H), lambda n: (n, 0)),
            pl.BlockSpec((BN, G_H), lambda n: (n, 0)),
        ],
        out_shape=[
            jax.ShapeDtypeStruct((N_TOK, L_H), bf16),
            jax.ShapeDtypeStruct((N_TOK, G_H), bf16),
        ],
    )(xb, local_mlp_weight_in.astype(bf16), local_mlp_weight_gate.astype(bf16),
      local_mlp_bias_in.reshape(1, L_H).astype(bf16),
      local_mlp_bias_gate.reshape(1, L_H).astype(bf16),
      global_mlp_weight_in.astype(bf16),
      global_mlp_bias_in.reshape(1, G_H).astype(bf16))

    out = pl.pallas_call(
        _combine_body,
        grid=(N_TOK // BN,),
        in_specs=[
            pl.BlockSpec((BN, L_H), lambda n: (n, 0)),
            pl.BlockSpec((BN, G_H), lambda n: (n, 0)),
            pl.BlockSpec((BN, F_DIM), lambda n: (n, 0)),
            full((L_H, D_OUT)),
            full((G_H, D_OUT)),
            full((F_DIM, D_OUT)),
            full((1, D_OUT)),
        ],
        out_specs=pl.BlockSpec((BN, D_OUT), lambda n: (n, 0)),
        out_shape=jax.ShapeDtypeStruct((N_TOK, D_OUT), f32),
    )(hl, gg, f, local_mlp_weight_out.astype(bf16),
      global_mlp_weight_out.astype(bf16), wc,
      (local_mlp_bias_out + global_mlp_bias_out).reshape(1, D_OUT))
    return out
